# baseline (speedup 1.0000x reference)
import functools

import jax
import jax.numpy as jnp
from jax import lax
from jax.experimental import pallas as pl
from jax.experimental.pallas import tpu as pltpu
from jax.experimental.pallas import tpu_sc as plsc

_NC = 2
_NS = 16
_NW = _NC * _NS
_CH = 64
_NB = 4
_IG = 4
_NIB = 3

_SLOW = 0
_G_SLOW = 16
_G_FAST = 64


def _cdiv(a, b):
    return (a + b - 1) // b


def _round_up(a, b):
    return _cdiv(a, b) * b


def _sc_mesh():
    return plsc.VectorSubcoreMesh(
        core_axis_name="c", subcore_axis_name="s",
        num_cores=_NC, num_subcores=_NS)


@functools.lru_cache(maxsize=None)
def _make_prop(n, n_pad, d, n_chunks):
    n_grp = n_chunks // _IG
    assert n_grp == _G_FAST and _G_SLOW % _NIB == _G_FAST % _NIB
    rpt = n_pad // _NS
    slab = _IG * _CH

    def body(ht_hbm, r_hbm, c_hbm, w_hbm, z_hbm, out_hbm,
             rows0, rows1, rows2, rows3,
             ri0, ri1, ri2, ci0, ci1, ci2, wi0, wi1, wi2, acc,
             gs0, gs1, gs2, gs3, ss0, ss1, ss2, ss3, is0, is1, is2):
        rows = [rows0, rows1, rows2, rows3]
        ribuf = [ri0, ri1, ri2]
        cibuf = [ci0, ci1, ci2]
        wxbuf = [wi0, wi1, wi2]
        gsem = [gs0, gs1, gs2, gs3]
        ssem = [ss0, ss1, ss2, ss3]
        isem = [is0, is1, is2]
        cid = lax.axis_index("c")
        sid = lax.axis_index("s")
        wid = cid * _NS + sid
        stripe = pl.ds(sid * rpt, rpt)
        pltpu.sync_copy(z_hbm.at[stripe], acc.at[stripe])
        pltpu.sync_copy(r_hbm.at[wid, pl.ds(0, _IG)], ribuf[0])
        pltpu.sync_copy(c_hbm.at[wid, pl.ds(0, _IG)], cibuf[0])
        pltpu.sync_copy(w_hbm.at[wid, pl.ds(0, _IG)], wxbuf[0])
        plsc.subcore_barrier()

        def idx_issue(g, ib):
            off = pl.ds(g * _IG, _IG)
            pltpu.async_copy(r_hbm.at[wid, off], ribuf[ib], isem[ib])
            pltpu.async_copy(c_hbm.at[wid, off], cibuf[ib], isem[ib])
            pltpu.async_copy(w_hbm.at[wid, off], wxbuf[ib], isem[ib])

        def idx_drain(ib):
            off = pl.ds(0, _IG)
            pltpu.make_async_copy(r_hbm.at[0, off], ribuf[ib], isem[ib]).wait()
            pltpu.make_async_copy(c_hbm.at[0, off], cibuf[ib], isem[ib]).wait()
            pltpu.make_async_copy(w_hbm.at[0, off], wxbuf[ib], isem[ib]).wait()

        def drain_rows(sem, b):
            pltpu.make_async_copy(z_hbm.at[pl.ds(0, _CH)], rows[b], sem).wait()

        def scale(wbuf, jj, b):
            def fgroup(g16, carry):
                w16 = wbuf[jj, pl.ds(g16 * 16, 16)]

                def fquad(q4, carry2):
                    for u in range(4):
                        lane = q4 * 4 + u
                        ws = w16.at[jnp.full((16,), lane, jnp.int32)].get(
                            mode="promise_in_bounds")
                        ei = g16 * 16 + lane
                        for q in range(d // 16):
                            sl = pl.ds(q * 16, 16)
                            rows[b][ei, sl] = rows[b][ei, sl] * ws
                    return carry2

                lax.fori_loop(0, 4, fquad, 0)
                return carry

            lax.fori_loop(0, _CH // 16, fgroup, 0)

        def group(g, ib, first=False, has_next=True):
            ibn = (ib + 1) % _NIB
            if has_next:
                idx_issue(g + 1, ibn)
            for jj in range(_IG):
                bp = (jj + 2) % _NB
                if not (first and jj < 2):
                    drain_rows(ssem[bp], bp)
                if jj == 2 and has_next:
                    idx_drain(ibn)
                if jj < 2:
                    pltpu.async_copy(ht_hbm.at[ribuf[ib].at[jj + 2]],
                                     rows[bp], gsem[bp])
                elif has_next:
                    pltpu.async_copy(ht_hbm.at[ribuf[ibn].at[jj - 2]],
                                     rows[bp], gsem[bp])
                drain_rows(gsem[jj], jj)
                scale(wxbuf[ib], jj, jj)
                pltpu.async_copy(rows[jj], acc.at[cibuf[ib].at[jj]],
                                 ssem[jj], add=True)

        pltpu.async_copy(ht_hbm.at[ribuf[0].at[0]], rows[0], gsem[0])
        pltpu.async_copy(ht_hbm.at[ribuf[0].at[1]], rows[1], gsem[1])
        group(0, 0, first=True)
        group(1, 1)

        def triple(t, carry):
            g = 2 + t * _NIB
            group(g + 0, 2)
            group(g + 1, 0)
            group(g + 2, 1)
            return carry

        tcnt = jnp.where(cid == _SLOW, (_G_SLOW - 4) // _NIB,
                         (_G_FAST - 4) // _NIB)
        lax.fori_loop(0, tcnt, triple, 0)

        ge = jnp.where(cid == _SLOW, _G_SLOW, _G_FAST)
        group(ge - 2, (_G_FAST - 2) % _NIB)
        group(ge - 1, (_G_FAST - 1) % _NIB, has_next=False)
        drain_rows(ssem[2], 2)
        drain_rows(ssem[3], 3)

        plsc.subcore_barrier()
        pltpu.sync_copy(acc.at[stripe], out_hbm.at[cid, stripe])

    return pl.kernel(
        body,
        out_type=jax.ShapeDtypeStruct((_NC, n_pad, d), jnp.float32),
        mesh=_sc_mesh(),
        compiler_params=pltpu.CompilerParams(use_tc_tiling_on_sc=False),
        scratch_types=(
            [pltpu.VMEM((_CH, d), jnp.float32) for _ in range(_NB)]
            + [pltpu.VMEM((_IG, _CH), jnp.int32) for _ in range(2 * _NIB)]
            + [pltpu.VMEM((_IG, _CH), jnp.float32) for _ in range(_NIB)]
            + [pltpu.VMEM_SHARED((n_pad, d), jnp.float32)]
            + [pltpu.SemaphoreType.DMA] * (2 * _NB + _NIB)
        ),
    )


@functools.lru_cache(maxsize=None)
def _make_deg(n_pad, n_chunks):
    n_grp = n_chunks // _IG
    assert n_grp == _G_FAST and _G_SLOW % _NIB == _G_FAST % _NIB
    rpt = n_pad // _NS
    slab = _IG * _CH

    def body(c_hbm, wx_hbm, z_hbm, out_hbm,
             ci0, ci1, ci2, wi0, wi1, wi2, acc,
             ss0, ss1, ss2, is0, is1, is2):
        cibuf = [ci0, ci1, ci2]
        wxbuf = [wi0, wi1, wi2]
        ssem = [ss0, ss1, ss2]
        isem = [is0, is1, is2]
        cid = lax.axis_index("c")
        sid = lax.axis_index("s")
        wid = cid * _NS + sid
        stripe = pl.ds(sid * rpt, rpt)
        pltpu.sync_copy(z_hbm.at[stripe], acc.at[stripe])
        pltpu.sync_copy(c_hbm.at[wid, pl.ds(0, _IG)], cibuf[0])
        pltpu.sync_copy(wx_hbm.at[wid, pl.ds(0, slab)], wxbuf[0])
        plsc.subcore_barrier()

        def idx_issue(g, ib):
            pltpu.async_copy(c_hbm.at[wid, pl.ds(g * _IG, _IG)],
                             cibuf[ib], isem[ib])
            pltpu.async_copy(wx_hbm.at[wid, pl.ds(g * slab, slab)],
                             wxbuf[ib], isem[ib])

        def idx_drain(ib):
            pltpu.make_async_copy(c_hbm.at[0, pl.ds(0, _IG)],
                                  cibuf[ib], isem[ib]).wait()
            pltpu.make_async_copy(wx_hbm.at[0, pl.ds(0, slab)],
                                  wxbuf[ib], isem[ib]).wait()

        def drain_scat(ib):
            for _ in range(_IG):
                pltpu.make_async_copy(wx_hbm.at[0, pl.ds(0, _CH)],
                                      wxbuf[ib].at[pl.ds(0, _CH)],
                                      ssem[ib]).wait()

        def group(g, ib, first=False, has_next=True, g_ge2=True):
            ibn = (ib + 1) % _NIB
            if not first:
                idx_drain(ib)
            if has_next:
                if g_ge2:
                    drain_scat(ibn)
                idx_issue(g + 1, ibn)
            for jj in range(_IG):
                pltpu.async_copy(wxbuf[ib].at[pl.ds(jj * _CH, _CH)],
                                 acc.at[cibuf[ib].at[jj]],
                                 ssem[ib], add=True)

        group(0, 0, first=True, g_ge2=False)
        group(1, 1, g_ge2=False)

        def triple(t, carry):
            g = 2 + t * _NIB
            group(g + 0, 2)
            group(g + 1, 0)
            group(g + 2, 1)
            return carry

        tcnt = jnp.where(cid == _SLOW, (_G_SLOW - 4) // _NIB,
                         (_G_FAST - 4) // _NIB)
        lax.fori_loop(0, tcnt, triple, 0)

        ge = jnp.where(cid == _SLOW, _G_SLOW, _G_FAST)
        group(ge - 2, (_G_FAST - 2) % _NIB)
        group(ge - 1, (_G_FAST - 1) % _NIB, has_next=False)
        for gg in (_G_FAST - 3, _G_FAST - 2, _G_FAST - 1):
            drain_scat(gg % _NIB)

        plsc.subcore_barrier()
        pltpu.sync_copy(acc.at[stripe], out_hbm.at[cid, stripe])

    return pl.kernel(
        body,
        out_type=jax.ShapeDtypeStruct((_NC, n_pad, 16), jnp.float32),
        mesh=_sc_mesh(),
        compiler_params=pltpu.CompilerParams(use_tc_tiling_on_sc=False),
        scratch_types=(
            [pltpu.VMEM((_IG, _CH), jnp.int32) for _ in range(_NIB)]
            + [pltpu.VMEM((_IG * _CH, 16), jnp.float32) for _ in range(_NIB)]
            + [pltpu.VMEM_SHARED((n_pad, 16), jnp.float32)]
            + [pltpu.SemaphoreType.DMA] * (2 * _NIB)
        ),
    )


_BLK = 512


def _tc_scaled_matmul(deg, x, w):
    n, k = x.shape
    m = w.shape[1]
    grid = _cdiv(n, _BLK)

    def body(deg_ref, x_ref, w_ref, o_ref):
        dis = lax.rsqrt(deg_ref[...])
        h = jnp.dot(x_ref[...], w_ref[...], preferred_element_type=jnp.float32)
        o_ref[...] = dis * h

    return pl.pallas_call(
        body,
        grid=(grid,),
        in_specs=[
            pl.BlockSpec((_BLK, 1), lambda i: (i, 0)),
            pl.BlockSpec((_BLK, k), lambda i: (i, 0)),
            pl.BlockSpec((k, m), lambda i: (0, 0)),
        ],
        out_specs=pl.BlockSpec((_BLK, m), lambda i: (i, 0)),
        out_shape=jax.ShapeDtypeStruct((n, m), jnp.float32),
    )(deg, x, w)


def _tc_combine_matmul(deg, a0, a1, hs, b, w):
    n, k = hs.shape
    m = w.shape[1]
    grid = _cdiv(n, _BLK)

    def body(deg_ref, a0_ref, a1_ref, hs_ref, b_ref, w_ref, o_ref):
        dis = lax.rsqrt(deg_ref[...])
        x1 = dis * (a0_ref[...] + a1_ref[...] + hs_ref[...]) + b_ref[...]
        x1 = jnp.maximum(x1, 0.0)
        h = jnp.dot(x1, w_ref[...], preferred_element_type=jnp.float32)
        o_ref[...] = dis * h

    blk = lambda i: (i, 0)
    return pl.pallas_call(
        body,
        grid=(grid,),
        in_specs=[
            pl.BlockSpec((_BLK, 1), blk),
            pl.BlockSpec((_BLK, k), blk),
            pl.BlockSpec((_BLK, k), blk),
            pl.BlockSpec((_BLK, k), blk),
            pl.BlockSpec((1, k), lambda i: (0, 0)),
            pl.BlockSpec((k, m), lambda i: (0, 0)),
        ],
        out_specs=pl.BlockSpec((_BLK, m), blk),
        out_shape=jax.ShapeDtypeStruct((n, m), jnp.float32),
    )(deg, a0, a1, hs, b, w)


def _tc_combine_normalize(deg, a0, a1, hs, b):
    n, m = hs.shape
    grid = _cdiv(n, _BLK)

    def body(deg_ref, a0_ref, a1_ref, hs_ref, b_ref, o_ref):
        dis = lax.rsqrt(deg_ref[...])
        x2 = dis * (a0_ref[...] + a1_ref[...] + hs_ref[...]) + b_ref[...]
        nrm = jnp.sqrt(jnp.sum(x2 * x2, axis=1, keepdims=True))
        o_ref[...] = x2 / jnp.maximum(nrm, 1e-12)

    blk = lambda i: (i, 0)
    return pl.pallas_call(
        body,
        grid=(grid,),
        in_specs=[
            pl.BlockSpec((_BLK, 1), blk),
            pl.BlockSpec((_BLK, m), blk),
            pl.BlockSpec((_BLK, m), blk),
            pl.BlockSpec((_BLK, m), blk),
            pl.BlockSpec((1, m), lambda i: (0, 0)),
        ],
        out_specs=pl.BlockSpec((_BLK, m), blk),
        out_shape=jax.ShapeDtypeStruct((n, m), jnp.float32),
    )(deg, a0, a1, hs, b)


def kernel(x, edge_index, edge_weight, W1, b1, W2, b2):
    n, nfeat = x.shape
    nhid = W1.shape[1]
    ncls = W2.shape[1]
    e = edge_index.shape[1]

    n_pad = _round_up(n, _NS * 8)
    cap_s = _G_SLOW * _IG * _CH
    cap_f = _G_FAST * _IG * _CH
    ns_tot = _NS * cap_s
    epw = cap_f
    n_chunks = epw // _CH

    def _split(a):
        a = jnp.pad(a, (0, ns_tot + _NS * cap_f - e))
        s = jnp.pad(a[:ns_tot].reshape(_NS, cap_s),
                    ((0, 0), (0, cap_f - cap_s)))
        f = a[ns_tot:].reshape(_NS, cap_f)
        parts = [s, f] if _SLOW == 0 else [f, s]
        return jnp.concatenate(parts, 0)

    sh = (_NW, n_chunks, _CH)
    r_p = _split(edge_index[0]).reshape(sh)
    c_p = _split(edge_index[1]).reshape(sh)
    w_p = _split(edge_weight).reshape(sh)
    wx_p = jnp.broadcast_to(
        _split(edge_weight).reshape(_NW, epw, 1), (_NW, epw, 16))

    degp = _make_deg(n_pad, n_chunks)(
        c_p, wx_p, jnp.zeros((n_pad, 16), jnp.float32))
    deg2d = 1.0 + degp[0, :n, :1] + degp[1, :n, :1]

    hs1 = _tc_scaled_matmul(deg2d, x, W1)
    agg1 = _make_prop(n, n_pad, nhid, n_chunks)(
        hs1, r_p, c_p, w_p, jnp.zeros((n_pad, nhid), jnp.float32))
    hs2 = _tc_combine_matmul(deg2d, agg1[0, :n], agg1[1, :n], hs1,
                             b1.reshape(1, nhid), W2)
    agg2 = _make_prop(n, n_pad, ncls, n_chunks)(
        hs2, r_p, c_p, w_p, jnp.zeros((n_pad, ncls), jnp.float32))
    return _tc_combine_normalize(deg2d, agg2[0, :n], agg2[1, :n], hs2,
                                 b2.reshape(1, ncls))


# speedup vs baseline: 1.0399x; 1.0399x over previous
import functools

import jax
import jax.numpy as jnp
from jax import lax
from jax.experimental import pallas as pl
from jax.experimental.pallas import tpu as pltpu
from jax.experimental.pallas import tpu_sc as plsc

_NC = 2
_NS = 16
_NW = _NC * _NS
_CH = 64
_NB = 4
_IG = 4
_NIB = 3

_SLOW = 1
_G_SLOW = 16
_G_FAST = 64


def _cdiv(a, b):
    return (a + b - 1) // b


def _round_up(a, b):
    return _cdiv(a, b) * b


def _sc_mesh():
    return plsc.VectorSubcoreMesh(
        core_axis_name="c", subcore_axis_name="s",
        num_cores=_NC, num_subcores=_NS)


@functools.lru_cache(maxsize=None)
def _make_prop(n, n_pad, d, n_chunks):
    n_grp = n_chunks // _IG
    assert n_grp == _G_FAST and _G_SLOW % _NIB == _G_FAST % _NIB
    rpt = n_pad // _NS
    slab = _IG * _CH

    def body(ht_hbm, r_hbm, c_hbm, w_hbm, z_hbm, out_hbm,
             rows0, rows1, rows2, rows3,
             ri0, ri1, ri2, ci0, ci1, ci2, wi0, wi1, wi2, acc,
             gs0, gs1, gs2, gs3, ss0, ss1, ss2, ss3, is0, is1, is2):
        rows = [rows0, rows1, rows2, rows3]
        ribuf = [ri0, ri1, ri2]
        cibuf = [ci0, ci1, ci2]
        wxbuf = [wi0, wi1, wi2]
        gsem = [gs0, gs1, gs2, gs3]
        ssem = [ss0, ss1, ss2, ss3]
        isem = [is0, is1, is2]
        cid = lax.axis_index("c")
        sid = lax.axis_index("s")
        wid = cid * _NS + sid
        stripe = pl.ds(sid * rpt, rpt)
        pltpu.sync_copy(z_hbm.at[stripe], acc.at[stripe])
        pltpu.sync_copy(r_hbm.at[wid, pl.ds(0, _IG)], ribuf[0])
        pltpu.sync_copy(c_hbm.at[wid, pl.ds(0, _IG)], cibuf[0])
        pltpu.sync_copy(w_hbm.at[wid, pl.ds(0, _IG)], wxbuf[0])
        plsc.subcore_barrier()

        def idx_issue(g, ib):
            off = pl.ds(g * _IG, _IG)
            pltpu.async_copy(r_hbm.at[wid, off], ribuf[ib], isem[ib])
            pltpu.async_copy(c_hbm.at[wid, off], cibuf[ib], isem[ib])
            pltpu.async_copy(w_hbm.at[wid, off], wxbuf[ib], isem[ib])

        def idx_drain(ib):
            off = pl.ds(0, _IG)
            pltpu.make_async_copy(r_hbm.at[0, off], ribuf[ib], isem[ib]).wait()
            pltpu.make_async_copy(c_hbm.at[0, off], cibuf[ib], isem[ib]).wait()
            pltpu.make_async_copy(w_hbm.at[0, off], wxbuf[ib], isem[ib]).wait()

        def drain_rows(sem, b):
            pltpu.make_async_copy(z_hbm.at[pl.ds(0, _CH)], rows[b], sem).wait()

        def scale(wbuf, jj, b):
            def fgroup(g16, carry):
                w16 = wbuf[jj, pl.ds(g16 * 16, 16)]

                def fquad(q4, carry2):
                    for u in range(4):
                        lane = q4 * 4 + u
                        ws = w16.at[jnp.full((16,), lane, jnp.int32)].get(
                            mode="promise_in_bounds")
                        ei = g16 * 16 + lane
                        for q in range(d // 16):
                            sl = pl.ds(q * 16, 16)
                            rows[b][ei, sl] = rows[b][ei, sl] * ws
                    return carry2

                lax.fori_loop(0, 4, fquad, 0)
                return carry

            lax.fori_loop(0, _CH // 16, fgroup, 0)

        def group(g, ib, first=False, has_next=True):
            ibn = (ib + 1) % _NIB
            if has_next:
                idx_issue(g + 1, ibn)
            for jj in range(_IG):
                bp = (jj + 2) % _NB
                if not (first and jj < 2):
                    drain_rows(ssem[bp], bp)
                if jj == 2 and has_next:
                    idx_drain(ibn)
                if jj < 2:
                    pltpu.async_copy(ht_hbm.at[ribuf[ib].at[jj + 2]],
                                     rows[bp], gsem[bp])
                elif has_next:
                    pltpu.async_copy(ht_hbm.at[ribuf[ibn].at[jj - 2]],
                                     rows[bp], gsem[bp])
                drain_rows(gsem[jj], jj)
                scale(wxbuf[ib], jj, jj)
                pltpu.async_copy(rows[jj], acc.at[cibuf[ib].at[jj]],
                                 ssem[jj], add=True)

        pltpu.async_copy(ht_hbm.at[ribuf[0].at[0]], rows[0], gsem[0])
        pltpu.async_copy(ht_hbm.at[ribuf[0].at[1]], rows[1], gsem[1])
        group(0, 0, first=True)
        group(1, 1)

        def triple(t, carry):
            g = 2 + t * _NIB
            group(g + 0, 2)
            group(g + 1, 0)
            group(g + 2, 1)
            return carry

        tcnt = jnp.where(cid == _SLOW, (_G_SLOW - 4) // _NIB,
                         (_G_FAST - 4) // _NIB)
        lax.fori_loop(0, tcnt, triple, 0)

        ge = jnp.where(cid == _SLOW, _G_SLOW, _G_FAST)
        group(ge - 2, (_G_FAST - 2) % _NIB)
        group(ge - 1, (_G_FAST - 1) % _NIB, has_next=False)
        drain_rows(ssem[2], 2)
        drain_rows(ssem[3], 3)

        plsc.subcore_barrier()
        pltpu.sync_copy(acc.at[stripe], out_hbm.at[cid, stripe])

    return pl.kernel(
        body,
        out_type=jax.ShapeDtypeStruct((_NC, n_pad, d), jnp.float32),
        mesh=_sc_mesh(),
        compiler_params=pltpu.CompilerParams(use_tc_tiling_on_sc=False),
        scratch_types=(
            [pltpu.VMEM((_CH, d), jnp.float32) for _ in range(_NB)]
            + [pltpu.VMEM((_IG, _CH), jnp.int32) for _ in range(2 * _NIB)]
            + [pltpu.VMEM((_IG, _CH), jnp.float32) for _ in range(_NIB)]
            + [pltpu.VMEM_SHARED((n_pad, d), jnp.float32)]
            + [pltpu.SemaphoreType.DMA] * (2 * _NB + _NIB)
        ),
    )


@functools.lru_cache(maxsize=None)
def _make_deg(n_pad, n_chunks):
    n_grp = n_chunks // _IG
    assert n_grp == _G_FAST and _G_SLOW % _NIB == _G_FAST % _NIB
    rpt = n_pad // _NS
    slab = _IG * _CH

    def body(c_hbm, wx_hbm, z_hbm, out_hbm,
             ci0, ci1, ci2, wi0, wi1, wi2, acc,
             ss0, ss1, ss2, is0, is1, is2):
        cibuf = [ci0, ci1, ci2]
        wxbuf = [wi0, wi1, wi2]
        ssem = [ss0, ss1, ss2]
        isem = [is0, is1, is2]
        cid = lax.axis_index("c")
        sid = lax.axis_index("s")
        wid = cid * _NS + sid
        stripe = pl.ds(sid * rpt, rpt)
        pltpu.sync_copy(z_hbm.at[stripe], acc.at[stripe])
        pltpu.sync_copy(c_hbm.at[wid, pl.ds(0, _IG)], cibuf[0])
        pltpu.sync_copy(wx_hbm.at[wid, pl.ds(0, slab)], wxbuf[0])
        plsc.subcore_barrier()

        def idx_issue(g, ib):
            pltpu.async_copy(c_hbm.at[wid, pl.ds(g * _IG, _IG)],
                             cibuf[ib], isem[ib])
            pltpu.async_copy(wx_hbm.at[wid, pl.ds(g * slab, slab)],
                             wxbuf[ib], isem[ib])

        def idx_drain(ib):
            pltpu.make_async_copy(c_hbm.at[0, pl.ds(0, _IG)],
                                  cibuf[ib], isem[ib]).wait()
            pltpu.make_async_copy(wx_hbm.at[0, pl.ds(0, slab)],
                                  wxbuf[ib], isem[ib]).wait()

        def drain_scat(ib):
            for _ in range(_IG):
                pltpu.make_async_copy(wx_hbm.at[0, pl.ds(0, _CH)],
                                      wxbuf[ib].at[pl.ds(0, _CH)],
                                      ssem[ib]).wait()

        def group(g, ib, first=False, has_next=True, g_ge2=True):
            ibn = (ib + 1) % _NIB
            if not first:
                idx_drain(ib)
            if has_next:
                if g_ge2:
                    drain_scat(ibn)
                idx_issue(g + 1, ibn)
            for jj in range(_IG):
                pltpu.async_copy(wxbuf[ib].at[pl.ds(jj * _CH, _CH)],
                                 acc.at[cibuf[ib].at[jj]],
                                 ssem[ib], add=True)

        group(0, 0, first=True, g_ge2=False)
        group(1, 1, g_ge2=False)

        def triple(t, carry):
            g = 2 + t * _NIB
            group(g + 0, 2)
            group(g + 1, 0)
            group(g + 2, 1)
            return carry

        tcnt = jnp.where(cid == _SLOW, (_G_SLOW - 4) // _NIB,
                         (_G_FAST - 4) // _NIB)
        lax.fori_loop(0, tcnt, triple, 0)

        ge = jnp.where(cid == _SLOW, _G_SLOW, _G_FAST)
        group(ge - 2, (_G_FAST - 2) % _NIB)
        group(ge - 1, (_G_FAST - 1) % _NIB, has_next=False)
        for gg in (_G_FAST - 3, _G_FAST - 2, _G_FAST - 1):
            drain_scat(gg % _NIB)

        plsc.subcore_barrier()
        pltpu.sync_copy(acc.at[stripe], out_hbm.at[cid, stripe])

    return pl.kernel(
        body,
        out_type=jax.ShapeDtypeStruct((_NC, n_pad, 16), jnp.float32),
        mesh=_sc_mesh(),
        compiler_params=pltpu.CompilerParams(use_tc_tiling_on_sc=False),
        scratch_types=(
            [pltpu.VMEM((_IG, _CH), jnp.int32) for _ in range(_NIB)]
            + [pltpu.VMEM((_IG * _CH, 16), jnp.float32) for _ in range(_NIB)]
            + [pltpu.VMEM_SHARED((n_pad, 16), jnp.float32)]
            + [pltpu.SemaphoreType.DMA] * (2 * _NIB)
        ),
    )


_BLK = 512


def _tc_scaled_matmul(deg, x, w):
    n, k = x.shape
    m = w.shape[1]
    grid = _cdiv(n, _BLK)

    def body(deg_ref, x_ref, w_ref, o_ref):
        dis = lax.rsqrt(deg_ref[...])
        h = jnp.dot(x_ref[...], w_ref[...], preferred_element_type=jnp.float32)
        o_ref[...] = dis * h

    return pl.pallas_call(
        body,
        grid=(grid,),
        in_specs=[
            pl.BlockSpec((_BLK, 1), lambda i: (i, 0)),
            pl.BlockSpec((_BLK, k), lambda i: (i, 0)),
            pl.BlockSpec((k, m), lambda i: (0, 0)),
        ],
        out_specs=pl.BlockSpec((_BLK, m), lambda i: (i, 0)),
        out_shape=jax.ShapeDtypeStruct((n, m), jnp.float32),
    )(deg, x, w)


def _tc_combine_matmul(deg, a0, a1, hs, b, w):
    n, k = hs.shape
    m = w.shape[1]
    grid = _cdiv(n, _BLK)

    def body(deg_ref, a0_ref, a1_ref, hs_ref, b_ref, w_ref, o_ref):
        dis = lax.rsqrt(deg_ref[...])
        x1 = dis * (a0_ref[...] + a1_ref[...] + hs_ref[...]) + b_ref[...]
        x1 = jnp.maximum(x1, 0.0)
        h = jnp.dot(x1, w_ref[...], preferred_element_type=jnp.float32)
        o_ref[...] = dis * h

    blk = lambda i: (i, 0)
    return pl.pallas_call(
        body,
        grid=(grid,),
        in_specs=[
            pl.BlockSpec((_BLK, 1), blk),
            pl.BlockSpec((_BLK, k), blk),
            pl.BlockSpec((_BLK, k), blk),
            pl.BlockSpec((_BLK, k), blk),
            pl.BlockSpec((1, k), lambda i: (0, 0)),
            pl.BlockSpec((k, m), lambda i: (0, 0)),
        ],
        out_specs=pl.BlockSpec((_BLK, m), blk),
        out_shape=jax.ShapeDtypeStruct((n, m), jnp.float32),
    )(deg, a0, a1, hs, b, w)


def _tc_combine_normalize(deg, a0, a1, hs, b):
    n, m = hs.shape
    grid = _cdiv(n, _BLK)

    def body(deg_ref, a0_ref, a1_ref, hs_ref, b_ref, o_ref):
        dis = lax.rsqrt(deg_ref[...])
        x2 = dis * (a0_ref[...] + a1_ref[...] + hs_ref[...]) + b_ref[...]
        nrm = jnp.sqrt(jnp.sum(x2 * x2, axis=1, keepdims=True))
        o_ref[...] = x2 / jnp.maximum(nrm, 1e-12)

    blk = lambda i: (i, 0)
    return pl.pallas_call(
        body,
        grid=(grid,),
        in_specs=[
            pl.BlockSpec((_BLK, 1), blk),
            pl.BlockSpec((_BLK, m), blk),
            pl.BlockSpec((_BLK, m), blk),
            pl.BlockSpec((_BLK, m), blk),
            pl.BlockSpec((1, m), lambda i: (0, 0)),
        ],
        out_specs=pl.BlockSpec((_BLK, m), blk),
        out_shape=jax.ShapeDtypeStruct((n, m), jnp.float32),
    )(deg, a0, a1, hs, b)


def kernel(x, edge_index, edge_weight, W1, b1, W2, b2):
    n, nfeat = x.shape
    nhid = W1.shape[1]
    ncls = W2.shape[1]
    e = edge_index.shape[1]

    n_pad = _round_up(n, _NS * 8)
    cap_s = _G_SLOW * _IG * _CH
    cap_f = _G_FAST * _IG * _CH
    ns_tot = _NS * cap_s
    epw = cap_f
    n_chunks = epw // _CH

    def _split(a):
        a = jnp.pad(a, (0, ns_tot + _NS * cap_f - e))
        s = jnp.pad(a[:ns_tot].reshape(_NS, cap_s),
                    ((0, 0), (0, cap_f - cap_s)))
        f = a[ns_tot:].reshape(_NS, cap_f)
        parts = [s, f] if _SLOW == 0 else [f, s]
        return jnp.concatenate(parts, 0)

    sh = (_NW, n_chunks, _CH)
    r_p = _split(edge_index[0]).reshape(sh)
    c_p = _split(edge_index[1]).reshape(sh)
    w_p = _split(edge_weight).reshape(sh)
    wx_p = jnp.broadcast_to(
        _split(edge_weight).reshape(_NW, epw, 1), (_NW, epw, 16))

    degp = _make_deg(n_pad, n_chunks)(
        c_p, wx_p, jnp.zeros((n_pad, 16), jnp.float32))
    deg2d = 1.0 + degp[0, :n, :1] + degp[1, :n, :1]

    hs1 = _tc_scaled_matmul(deg2d, x, W1)
    agg1 = _make_prop(n, n_pad, nhid, n_chunks)(
        hs1, r_p, c_p, w_p, jnp.zeros((n_pad, nhid), jnp.float32))
    hs2 = _tc_combine_matmul(deg2d, agg1[0, :n], agg1[1, :n], hs1,
                             b1.reshape(1, nhid), W2)
    agg2 = _make_prop(n, n_pad, ncls, n_chunks)(
        hs2, r_p, c_p, w_p, jnp.zeros((n_pad, ncls), jnp.float32))
    return _tc_combine_normalize(deg2d, agg2[0, :n], agg2[1, :n], hs2,
                                 b2.reshape(1, ncls))


# speedup vs baseline: 1.4500x; 1.3944x over previous
import functools

import jax
import jax.numpy as jnp
from jax import lax
from jax.experimental import pallas as pl
from jax.experimental.pallas import tpu as pltpu
from jax.experimental.pallas import tpu_sc as plsc

_NC = 2
_NS = 16
_NW = _NC * _NS
_CH = 64
_NB = 4
_IG = 4
_NIB = 3


def _cdiv(a, b):
    return (a + b - 1) // b


def _round_up(a, b):
    return _cdiv(a, b) * b


def _sc_mesh():
    return plsc.VectorSubcoreMesh(
        core_axis_name="c", subcore_axis_name="s",
        num_cores=_NC, num_subcores=_NS)


@functools.lru_cache(maxsize=None)
def _make_prop(n, n_pad, d, n_chunks):
    n_grp = n_chunks // _IG
    assert n_chunks % _IG == 0 and n_grp >= 4 and (n_grp - 4) % _NIB == 0
    rpt = n_pad // _NS
    slab = _IG * _CH

    def body(ht_hbm, r_hbm, c_hbm, w_hbm, z_hbm, out_hbm,
             rows0, rows1, rows2, rows3,
             ri0, ri1, ri2, ci0, ci1, ci2, wi0, wi1, wi2, acc,
             gs0, gs1, gs2, gs3, ss0, ss1, ss2, ss3, is0, is1, is2):
        rows = [rows0, rows1, rows2, rows3]
        ribuf = [ri0, ri1, ri2]
        cibuf = [ci0, ci1, ci2]
        wxbuf = [wi0, wi1, wi2]
        gsem = [gs0, gs1, gs2, gs3]
        ssem = [ss0, ss1, ss2, ss3]
        isem = [is0, is1, is2]
        cid = lax.axis_index("c")
        sid = lax.axis_index("s")
        wid = cid * _NS + sid
        stripe = pl.ds(sid * rpt, rpt)
        pltpu.sync_copy(z_hbm.at[stripe], acc.at[stripe])
        pltpu.sync_copy(r_hbm.at[wid, pl.ds(0, _IG)], ribuf[0])
        pltpu.sync_copy(c_hbm.at[wid, pl.ds(0, _IG)], cibuf[0])
        pltpu.sync_copy(w_hbm.at[wid, pl.ds(0, _IG)], wxbuf[0])
        plsc.subcore_barrier()

        def idx_issue(g, ib):
            off = pl.ds(g * _IG, _IG)
            pltpu.async_copy(r_hbm.at[wid, off], ribuf[ib], isem[ib])
            pltpu.async_copy(c_hbm.at[wid, off], cibuf[ib], isem[ib])
            pltpu.async_copy(w_hbm.at[wid, off], wxbuf[ib], isem[ib])

        def idx_drain(ib):
            off = pl.ds(0, _IG)
            pltpu.make_async_copy(r_hbm.at[0, off], ribuf[ib], isem[ib]).wait()
            pltpu.make_async_copy(c_hbm.at[0, off], cibuf[ib], isem[ib]).wait()
            pltpu.make_async_copy(w_hbm.at[0, off], wxbuf[ib], isem[ib]).wait()

        def drain_rows(sem, b):
            pltpu.make_async_copy(z_hbm.at[pl.ds(0, _CH)], rows[b], sem).wait()

        def scale(wbuf, jj, b):
            def fgroup(g16, carry):
                w16 = wbuf[jj, pl.ds(g16 * 16, 16)]

                def fquad(q4, carry2):
                    for u in range(4):
                        lane = q4 * 4 + u
                        ws = w16.at[jnp.full((16,), lane, jnp.int32)].get(
                            mode="promise_in_bounds")
                        ei = g16 * 16 + lane
                        for q in range(d // 16):
                            sl = pl.ds(q * 16, 16)
                            rows[b][ei, sl] = rows[b][ei, sl] * ws
                    return carry2

                lax.fori_loop(0, 4, fquad, 0)
                return carry

            lax.fori_loop(0, _CH // 16, fgroup, 0)

        def group(g, ib, first=False, has_next=True):
            ibn = (ib + 1) % _NIB
            if has_next:
                idx_issue(g + 1, ibn)
            for jj in range(_IG):
                bp = (jj + 2) % _NB
                if not (first and jj < 2):
                    drain_rows(ssem[bp], bp)
                if jj == 2 and has_next:
                    idx_drain(ibn)
                if jj < 2:
                    pltpu.async_copy(ht_hbm.at[ribuf[ib].at[jj + 2]],
                                     rows[bp], gsem[bp])
                elif has_next:
                    pltpu.async_copy(ht_hbm.at[ribuf[ibn].at[jj - 2]],
                                     rows[bp], gsem[bp])
                drain_rows(gsem[jj], jj)
                scale(wxbuf[ib], jj, jj)
                pltpu.async_copy(rows[jj], acc.at[cibuf[ib].at[jj]],
                                 ssem[jj], add=True)

        pltpu.async_copy(ht_hbm.at[ribuf[0].at[0]], rows[0], gsem[0])
        pltpu.async_copy(ht_hbm.at[ribuf[0].at[1]], rows[1], gsem[1])
        group(0, 0, first=True)
        group(1, 1)

        def triple(t, carry):
            g = 2 + t * _NIB
            group(g + 0, 2)
            group(g + 1, 0)
            group(g + 2, 1)
            return carry

        lax.fori_loop(0, (n_grp - 4) // _NIB, triple, 0)

        group(n_grp - 2, (n_grp - 2) % _NIB)
        group(n_grp - 1, (n_grp - 1) % _NIB, has_next=False)
        drain_rows(ssem[2], 2)
        drain_rows(ssem[3], 3)

        plsc.subcore_barrier()
        pltpu.sync_copy(acc.at[stripe], out_hbm.at[cid, stripe])

    return pl.kernel(
        body,
        out_type=jax.ShapeDtypeStruct((_NC, n_pad, d), jnp.float32),
        mesh=_sc_mesh(),
        compiler_params=pltpu.CompilerParams(use_tc_tiling_on_sc=False),
        scratch_types=(
            [pltpu.VMEM((_CH, d), jnp.float32) for _ in range(_NB)]
            + [pltpu.VMEM((_IG, _CH), jnp.int32) for _ in range(2 * _NIB)]
            + [pltpu.VMEM((_IG, _CH), jnp.float32) for _ in range(_NIB)]
            + [pltpu.VMEM_SHARED((n_pad, d), jnp.float32)]
            + [pltpu.SemaphoreType.DMA] * (2 * _NB + _NIB)
        ),
    )


_BLK = 512


def _tc_scaled_matmul(deg, x, w):
    n, k = x.shape
    m = w.shape[1]
    grid = _cdiv(n, _BLK)

    def body(deg_ref, x_ref, w_ref, o_ref):
        dis = lax.rsqrt(deg_ref[...])
        h = jnp.dot(x_ref[...], w_ref[...], preferred_element_type=jnp.float32)
        o_ref[...] = dis * h

    return pl.pallas_call(
        body,
        grid=(grid,),
        in_specs=[
            pl.BlockSpec((_BLK, 1), lambda i: (i, 0)),
            pl.BlockSpec((_BLK, k), lambda i: (i, 0)),
            pl.BlockSpec((k, m), lambda i: (0, 0)),
        ],
        out_specs=pl.BlockSpec((_BLK, m), lambda i: (i, 0)),
        out_shape=jax.ShapeDtypeStruct((n, m), jnp.float32),
    )(deg, x, w)


def _tc_combine_matmul(deg, a0, a1, hs, b, w):
    n, k = hs.shape
    m = w.shape[1]
    grid = _cdiv(n, _BLK)

    def body(deg_ref, a0_ref, a1_ref, hs_ref, b_ref, w_ref, o_ref):
        dis = lax.rsqrt(deg_ref[...])
        x1 = dis * (a0_ref[...] + a1_ref[...] + hs_ref[...]) + b_ref[...]
        x1 = jnp.maximum(x1, 0.0)
        h = jnp.dot(x1, w_ref[...], preferred_element_type=jnp.float32)
        o_ref[...] = dis * h

    blk = lambda i: (i, 0)
    return pl.pallas_call(
        body,
        grid=(grid,),
        in_specs=[
            pl.BlockSpec((_BLK, 1), blk),
            pl.BlockSpec((_BLK, k), blk),
            pl.BlockSpec((_BLK, k), blk),
            pl.BlockSpec((_BLK, k), blk),
            pl.BlockSpec((1, k), lambda i: (0, 0)),
            pl.BlockSpec((k, m), lambda i: (0, 0)),
        ],
        out_specs=pl.BlockSpec((_BLK, m), blk),
        out_shape=jax.ShapeDtypeStruct((n, m), jnp.float32),
    )(deg, a0, a1, hs, b, w)


def _tc_combine_normalize(deg, a0, a1, hs, b):
    n, m = hs.shape
    grid = _cdiv(n, _BLK)

    def body(deg_ref, a0_ref, a1_ref, hs_ref, b_ref, o_ref):
        dis = lax.rsqrt(deg_ref[...])
        x2 = dis * (a0_ref[...] + a1_ref[...] + hs_ref[...]) + b_ref[...]
        nrm = jnp.sqrt(jnp.sum(x2 * x2, axis=1, keepdims=True))
        o_ref[...] = x2 / jnp.maximum(nrm, 1e-12)

    blk = lambda i: (i, 0)
    return pl.pallas_call(
        body,
        grid=(grid,),
        in_specs=[
            pl.BlockSpec((_BLK, 1), blk),
            pl.BlockSpec((_BLK, m), blk),
            pl.BlockSpec((_BLK, m), blk),
            pl.BlockSpec((_BLK, m), blk),
            pl.BlockSpec((1, m), lambda i: (0, 0)),
        ],
        out_specs=pl.BlockSpec((_BLK, m), blk),
        out_shape=jax.ShapeDtypeStruct((n, m), jnp.float32),
    )(deg, a0, a1, hs, b)


def kernel(x, edge_index, edge_weight, W1, b1, W2, b2):
    n, nfeat = x.shape
    nhid = W1.shape[1]
    ncls = W2.shape[1]
    e = edge_index.shape[1]

    n_pad = _round_up(n, _NS * 8)
    epw = _round_up(_cdiv(e, _NW), _IG * _CH)
    while ((epw // (_IG * _CH)) - 4) % _NIB:
        epw += _IG * _CH
    e_pad = epw * _NW
    n_chunks = epw // _CH

    pad = e_pad - e
    sh = (_NW, n_chunks, _CH)
    r_p = jnp.pad(edge_index[0], (0, pad)).reshape(sh)
    c_p = jnp.pad(edge_index[1], (0, pad)).reshape(sh)
    w_p = jnp.pad(edge_weight, (0, pad)).reshape(sh)

    degp = _make_prop(n, n_pad, 16, n_chunks)(
        jnp.ones((n, 16), jnp.float32), r_p, c_p, w_p,
        jnp.zeros((n_pad, 16), jnp.float32))
    deg2d = 1.0 + degp[0, :n, :1] + degp[1, :n, :1]

    hs1 = _tc_scaled_matmul(deg2d, x, W1)
    agg1 = _make_prop(n, n_pad, nhid, n_chunks)(
        hs1, r_p, c_p, w_p, jnp.zeros((n_pad, nhid), jnp.float32))
    hs2 = _tc_combine_matmul(deg2d, agg1[0, :n], agg1[1, :n], hs1,
                             b1.reshape(1, nhid), W2)
    agg2 = _make_prop(n, n_pad, ncls, n_chunks)(
        hs2, r_p, c_p, w_p, jnp.zeros((n_pad, ncls), jnp.float32))
    return _tc_combine_normalize(deg2d, agg2[0, :n], agg2[1, :n], hs2,
                                 b2.reshape(1, ncls))
